# baseline (speedup 1.0000x reference)
import jax
import jax.numpy as jnp
from jax import lax
from jax.experimental import pallas as pl
from jax.experimental.pallas import tpu as pltpu
from jax.experimental.pallas import tpu_sc as plsc

_B = 4
_S = 2048
_D = 1024
_NC = 2
_NS = 16
_NW = _NC * _NS
_SPW = _S // _NW
_CH = 8
_NQ = _SPW // _CH
_NR = 3
_QR = _B * _CH
_VPQ = _CH * _D // 16


def _body(x_hbm, tok_hbm, pos_hbm, out_hbm,
          idx_q, p0, p1, p2, g0, g1, g2,
          isem, psem0, psem1, psem2, gsem0, gsem1, gsem2,
          ssem0, ssem1, ssem2):
    pbufs = (p0, p1, p2)
    gbufs = (g0, g1, g2)
    psems = (psem0, psem1, psem2)
    gsems = (gsem0, gsem1, gsem2)
    ssems = (ssem0, ssem1, ssem2)

    wid = lax.axis_index("s") * _NC + lax.axis_index("c")
    s0 = wid * _SPW

    ih = [pltpu.async_copy(x_hbm.at[b, pl.ds(s0 + q * _CH, _CH)],
                           idx_q.at[pl.ds(q * _QR + b * _CH, _CH)], isem)
          for q in range(_NQ) for b in range(_B)]
    for h in ih:
        h.wait()

    def start_pos(q):
        k = q % _NR
        return pltpu.async_copy(
            pos_hbm.at[pl.ds(s0 + q * _CH, _CH)], pbufs[k], psems[k])

    def start_quad_gather(q):
        k = q % _NR
        return pltpu.async_copy(
            tok_hbm.at[idx_q.at[pl.ds(q * _QR, _QR)]], gbufs[k], gsems[k])

    ph = [start_pos(q) for q in range(_NR)]
    gh = [start_quad_gather(q) for q in range(_NR)]
    sh = [None] * _NR

    for q in range(_NQ):
        k = q % _NR
        ph[k].wait()
        gh[k].wait()
        if q >= 1:
            kp = (q - 1) % _NR
            for h in sh[kp]:
                h.wait()
            sh[kp] = None
            if q + 2 < _NQ:
                ph[kp] = start_pos(q + 2)
                gh[kp] = start_quad_gather(q + 2)
        pbuf = pbufs[k]
        buf = gbufs[k]

        @plsc.parallel_loop(0, _VPQ, unroll=2)
        def _(i, buf=buf, pbuf=pbuf):
            r = i >> 6
            c = pl.multiple_of((i & 63) << 4, 16)
            pv = pbuf[r, pl.ds(c, 16)]
            for b in range(_B):
                buf[b * _CH + r, pl.ds(c, 16)] += pv

        sh[k] = [
            pltpu.async_copy(
                buf.at[pl.ds(b * _CH, _CH)],
                out_hbm.at[pl.ds(b * _S + s0 + q * _CH, _CH)], ssems[k])
            for b in range(_B)
        ]

    for k in range(_NR):
        if sh[k] is not None:
            for h in sh[k]:
                h.wait()


def kernel(x, tok_table, pos_table):
    mesh = plsc.VectorSubcoreMesh(core_axis_name="c", subcore_axis_name="s")
    out = pl.kernel(
        _body,
        out_type=jax.ShapeDtypeStruct((_B * _S, _D), jnp.float32),
        mesh=mesh,
        scratch_types=(
            [pltpu.VMEM((_B * _SPW,), jnp.int32)]
            + [pltpu.VMEM((_CH, _D), jnp.float32)] * 3
            + [pltpu.VMEM((_QR, _D), jnp.float32)] * _NR
            + [pltpu.SemaphoreType.DMA] * 10
        ),
    )(x.astype(jnp.int32), tok_table, pos_table)
    return out.reshape(_B, _S, _D)


# speedup vs baseline: 1.0459x; 1.0459x over previous
import jax
import jax.numpy as jnp
from jax import lax
from jax.experimental import pallas as pl
from jax.experimental.pallas import tpu as pltpu
from jax.experimental.pallas import tpu_sc as plsc

_B = 4
_S = 2048
_D = 1024
_NC = 2
_NS = 16
_NW = _NC * _NS
_SPW = _S // _NW
_CH = 8
_NQ = _SPW // _CH
_NR = 3
_VPC = _CH * _D // 16


def _body(x_hbm, tok_hbm, pos_hbm, out_hbm,
          idx_v, p0, p1, p2,
          g00, g01, g02, g03, g10, g11, g12, g13, g20, g21, g22, g23,
          isem, psem0, psem1, psem2, gsem0, gsem1, gsem2,
          ssem0, ssem1, ssem2):
    pbufs = (p0, p1, p2)
    gbufs = ((g00, g01, g02, g03), (g10, g11, g12, g13), (g20, g21, g22, g23))
    psems = (psem0, psem1, psem2)
    gsems = (gsem0, gsem1, gsem2)
    ssems = (ssem0, ssem1, ssem2)

    wid = lax.axis_index("s") * _NC + lax.axis_index("c")
    s0 = wid * _SPW

    def start_pos(q):
        k = q % _NR
        return pltpu.async_copy(
            pos_hbm.at[pl.ds(s0 + q * _CH, _CH)], pbufs[k], psems[k])

    def start_quad_gathers(q):
        k = q % _NR
        return [
            pltpu.async_copy(
                tok_hbm.at[idx_v.at[b, pl.ds(q * _CH, _CH)]],
                gbufs[k][b], gsems[k])
            for b in range(_B)
        ]

    ph = [start_pos(q) for q in range(_NR)]

    ih = [pltpu.async_copy(x_hbm.at[b, pl.ds(s0, _SPW)], idx_v.at[b], isem)
          for b in range(_B)]
    for h in ih:
        h.wait()

    gh = [start_quad_gathers(q) for q in range(_NR)]
    sh = [None] * _NR

    def add_half(buf_pair, pbuf):
        @plsc.parallel_loop(0, _VPC, unroll=2)
        def _(i, buf_pair=buf_pair, pbuf=pbuf):
            r = i >> 6
            c = pl.multiple_of((i & 63) << 4, 16)
            pv = pbuf[r, pl.ds(c, 16)]
            for buf in buf_pair:
                buf[r, pl.ds(c, 16)] += pv

    for q in range(_NQ):
        k = q % _NR
        ph[k].wait()
        for h in gh[k]:
            h.wait()
        pbuf = pbufs[k]
        bufs = gbufs[k]

        def scat(b):
            return pltpu.async_copy(
                bufs[b], out_hbm.at[pl.ds(b * _S + s0 + q * _CH, _CH)],
                ssems[k])

        add_half(bufs[:2], pbuf)
        s01 = [scat(0), scat(1)]
        add_half(bufs[2:], pbuf)
        sh[k] = s01 + [scat(2), scat(3)]

        if q + _NR < _NQ:
            for h in sh[k]:
                h.wait()
            ph[k] = start_pos(q + _NR)
            gh[k] = start_quad_gathers(q + _NR)

    for k in range(_NR):
        if sh[k] is not None:
            for h in sh[k]:
                h.wait()


def kernel(x, tok_table, pos_table):
    mesh = plsc.VectorSubcoreMesh(core_axis_name="c", subcore_axis_name="s")
    out = pl.kernel(
        _body,
        out_type=jax.ShapeDtypeStruct((_B * _S, _D), jnp.float32),
        mesh=mesh,
        scratch_types=(
            [pltpu.VMEM((_B, _SPW), jnp.int32)]
            + [pltpu.VMEM((_CH, _D), jnp.float32)] * (3 + _NR * _B)
            + [pltpu.SemaphoreType.DMA] * 10
        ),
    )(x.astype(jnp.int32), tok_table, pos_table)
    return out.reshape(_B, _S, _D)


# speedup vs baseline: 1.0548x; 1.0085x over previous
import jax
import jax.numpy as jnp
from jax import lax
from jax.experimental import pallas as pl
from jax.experimental.pallas import tpu as pltpu
from jax.experimental.pallas import tpu_sc as plsc

_B = 4
_S = 2048
_D = 1024
_NC = 2
_NS = 16
_NW = _NC * _NS
_SPW = _S // _NW
_CH = 8
_NQ = _SPW // _CH
_NR = 3
_VPQ = _CH * _D // 16


def _body(x_hbm, tok_hbm, pos_hbm, out_hbm,
          idx_v, p0, p1, p2,
          g00, g01, g02, g03, g10, g11, g12, g13, g20, g21, g22, g23,
          isem, psem0, psem1, psem2, gsem0, gsem1, gsem2,
          ssem0, ssem1, ssem2):
    pbufs = (p0, p1, p2)
    gbufs = ((g00, g01, g02, g03), (g10, g11, g12, g13), (g20, g21, g22, g23))
    psems = (psem0, psem1, psem2)
    gsems = (gsem0, gsem1, gsem2)
    ssems = (ssem0, ssem1, ssem2)

    wid = lax.axis_index("s") * _NC + lax.axis_index("c")
    s0 = wid * _SPW

    ih = [pltpu.async_copy(x_hbm.at[b, pl.ds(s0, _SPW)], idx_v.at[b], isem)
          for b in range(_B)]
    for h in ih:
        h.wait()

    def start_pos(q):
        k = q % _NR
        return pltpu.async_copy(
            pos_hbm.at[pl.ds(s0 + q * _CH, _CH)], pbufs[k], psems[k])

    def start_quad_gathers(q):
        k = q % _NR
        return [
            pltpu.async_copy(
                tok_hbm.at[idx_v.at[b, pl.ds(q * _CH, _CH)]],
                gbufs[k][b], gsems[k])
            for b in range(_B)
        ]

    ph = [start_pos(q) for q in range(_NR)]
    gh = [start_quad_gathers(q) for q in range(_NR)]
    sh = [None] * _NR

    for q in range(_NQ):
        k = q % _NR
        ph[k].wait()
        for h in gh[k]:
            h.wait()
        pbuf = pbufs[k]
        bufs = gbufs[k]

        @plsc.parallel_loop(0, _VPQ, unroll=2)
        def _(i, bufs=bufs, pbuf=pbuf):
            r = i >> 6
            c = pl.multiple_of((i & 63) << 4, 16)
            pv = pbuf[r, pl.ds(c, 16)]
            for b in range(_B):
                bufs[b][r, pl.ds(c, 16)] += pv

        sh[k] = [
            pltpu.async_copy(
                bufs[b], out_hbm.at[pl.ds(b * _S + s0 + q * _CH, _CH)],
                ssems[k])
            for b in range(_B)
        ]
        if q + _NR < _NQ:
            for h in sh[k]:
                h.wait()
            ph[k] = start_pos(q + _NR)
            gh[k] = start_quad_gathers(q + _NR)

    for k in range(_NR):
        if sh[k] is not None:
            for h in sh[k]:
                h.wait()


def kernel(x, tok_table, pos_table):
    mesh = plsc.VectorSubcoreMesh(core_axis_name="c", subcore_axis_name="s")
    out = pl.kernel(
        _body,
        out_type=jax.ShapeDtypeStruct((_B * _S, _D), jnp.float32),
        mesh=mesh,
        scratch_types=(
            [pltpu.VMEM((_B, _SPW), jnp.int32)]
            + [pltpu.VMEM((_CH, _D), jnp.float32)] * (3 + _NR * _B)
            + [pltpu.SemaphoreType.DMA] * 10
        ),
    )(x.astype(jnp.int32), tok_table, pos_table)
    return out.reshape(_B, _S, _D)
